# baseline (speedup 1.0000x reference)
import functools

import jax
import jax.numpy as jnp
from jax import lax
from jax.experimental import pallas as pl
from jax.experimental.pallas import tpu as pltpu
from jax.experimental.pallas import tpu_sc as plsc

HIST = 50
BATCH = 16384
D = 64
NC, NS = 2, 16
NW = NC * NS
BBLK = 128
B_PER_W = BATCH // NW
NBB = B_PER_W // BBLK
HBLKS = (HIST + 7) // 8
NG = BBLK // 16


def _make_gather():
    mesh = plsc.VectorSubcoreMesh(core_axis_name="c", subcore_axis_name="s")

    @functools.partial(
        pl.kernel,
        mesh=mesh,
        out_type=jax.ShapeDtypeStruct((HIST, D, BATCH), jnp.float32),
        scratch_types=[
            pltpu.VMEM((8, BBLK), jnp.int32),
            pltpu.VMEM((8, BBLK), jnp.int32),
            pltpu.VMEM((4, BBLK, 128), jnp.float32),
            pltpu.VMEM((D, BBLK), jnp.float32),
            pltpu.SemaphoreType.DMA,
        ],
        compiler_params=pltpu.CompilerParams(
            use_tc_tiling_on_sc=True, needs_layout_passes=False
        ),
    )
    def gather(ids_hbm, table_hbm, out_hbm, pair_v, half_v, stage_v, blk_v, sem):
        wid = lax.axis_index("s") * NC + lax.axis_index("c")
        bbase = wid * B_PER_W
        lane = lax.iota(jnp.int32, 16)

        HB = BBLK // 2

        def start_gather(r):
            buf = stage_v.at[r % 4]
            pltpu.async_copy(
                table_hbm.at[pair_v.at[r, pl.ds(0, HB)]],
                buf.at[pl.ds(0, HB)], sem,
            )
            pltpu.async_copy(
                table_hbm.at[pair_v.at[r, pl.ds(HB, HB)]],
                buf.at[pl.ds(HB, HB)], sem,
            )

        def wait_gather(r):
            pltpu.make_async_copy(
                table_hbm.at[pl.ds(0, BBLK)], stage_v.at[r % 4], sem
            ).wait()

        def bb_body(bb, carry):
            b0 = pl.multiple_of(bbase + bb * BBLK, BBLK)
            for hblk in range(HBLKS):
                nh = min(8, HIST - hblk * 8)
                pltpu.sync_copy(
                    ids_hbm.at[pl.ds(hblk * 8, nh), pl.ds(b0, BBLK)],
                    pair_v.at[pl.ds(0, nh)],
                )

                def prep_body(r, c):
                    for g in range(NG):
                        ids16 = pair_v[r, pl.ds(g * 16, 16)]
                        half_v[r, pl.ds(g * 16, 16)] = (ids16 & 1) << 6
                        pair_v[r, pl.ds(g * 16, 16)] = ids16 >> 1
                    return c

                lax.fori_loop(0, nh, prep_body, 0)
                for rr in range(min(3, nh)):
                    start_gather(rr)

                def unit_body(r, c):
                    h = hblk * 8 + r
                    wait_gather(r)

                    @pl.when(r < nh - 3)
                    def _():
                        start_gather(r + 3)

                    stage = stage_v.at[r % 4]
                    zeros = lane * 0
                    bases = []
                    for g in range(NG):
                        rows128 = (lane + g * 16) << 7
                        half16 = half_v[r, pl.ds(g * 16, 16)]
                        bases.append(rows128 + half16)

                    vq0 = tuple(
                        plsc.load_gather(stage, [zeros, bases[g]])
                        for g in range(NG)
                    )

                    def d_body(d, vq):
                        new = tuple(
                            plsc.load_gather(stage, [zeros, bases[g] + d])
                            for g in range(NG)
                        )
                        for g in range(NG):
                            blk_v[d - 1, pl.ds(g * 16, 16)] = vq[g]
                        return new

                    vq_last = lax.fori_loop(1, D, d_body, vq0)
                    for g in range(NG):
                        blk_v[D - 1, pl.ds(g * 16, 16)] = vq_last[g]
                    pltpu.sync_copy(blk_v, out_hbm.at[h, :, pl.ds(b0, BBLK)])
                    return c

                lax.fori_loop(0, nh, unit_body, 0)
            return carry

        lax.fori_loop(0, NBB, bb_body, 0)

    return gather


def kernel(input_ids, table):
    ids_t = jnp.transpose(input_ids).astype(jnp.int32)
    table2 = table.reshape(table.shape[0] // 2, 2 * D)
    out_t = _make_gather()(ids_t, table2)
    return jnp.transpose(out_t, (2, 0, 1))


# speedup vs baseline: 1.1578x; 1.1578x over previous
import functools

import jax
import jax.numpy as jnp
from jax import lax
from jax.experimental import pallas as pl
from jax.experimental.pallas import tpu as pltpu
from jax.experimental.pallas import tpu_sc as plsc

HIST = 50
BATCH = 16384
D = 64
NC, NS = 2, 16
NW = NC * NS
BBLK = 128
B_PER_W = BATCH // NW
NBB = B_PER_W // BBLK
NG = BBLK // 16
HB = BBLK // 2


def _make_gather():
    mesh = plsc.VectorSubcoreMesh(core_axis_name="c", subcore_axis_name="s")

    @functools.partial(
        pl.kernel,
        mesh=mesh,
        out_type=jax.ShapeDtypeStruct((HIST, D, BATCH), jnp.float32),
        scratch_types=[
            pltpu.VMEM((8, BBLK), jnp.int32),
            pltpu.VMEM((8, BBLK), jnp.int32),
            pltpu.VMEM((4, BBLK, 128), jnp.float32),
            pltpu.VMEM((D, BBLK), jnp.float32),
            pltpu.SemaphoreType.DMA,
        ],
        compiler_params=pltpu.CompilerParams(
            use_tc_tiling_on_sc=True, needs_layout_passes=False
        ),
    )
    def gather(ids_hbm, table_hbm, out_hbm, pair_v, half_v, stage_v, blk_v, sem):
        wid = lax.axis_index("s") * NC + lax.axis_index("c")
        bbase = wid * B_PER_W
        lane = lax.iota(jnp.int32, 16)
        zeros = lane * 0

        def start_gather(r):
            buf = stage_v.at[r % 4]
            pltpu.async_copy(
                table_hbm.at[pair_v.at[r, pl.ds(0, HB)]],
                buf.at[pl.ds(0, HB)], sem,
            )
            pltpu.async_copy(
                table_hbm.at[pair_v.at[r, pl.ds(HB, HB)]],
                buf.at[pl.ds(HB, HB)], sem,
            )

        def wait_gather(r):
            pltpu.make_async_copy(
                table_hbm.at[pl.ds(0, BBLK)], stage_v.at[r % 4], sem
            ).wait()

        def run_hblk(h0, nh, b0):
            pltpu.sync_copy(
                ids_hbm.at[pl.ds(h0, nh), pl.ds(b0, BBLK)],
                pair_v.at[pl.ds(0, nh)],
            )

            def prep_body(r, c):
                for g in range(NG):
                    ids16 = pair_v[r, pl.ds(g * 16, 16)]
                    half_v[r, pl.ds(g * 16, 16)] = (ids16 & 1) << 6
                    pair_v[r, pl.ds(g * 16, 16)] = ids16 >> 1
                return c

            lax.fori_loop(0, nh, prep_body, 0)
            for rr in range(min(3, nh)):
                start_gather(rr)

            def unit_body(r, c):
                h = h0 + r
                wait_gather(r)

                @pl.when(r < nh - 3)
                def _():
                    start_gather(r + 3)

                stage = stage_v.at[r % 4]
                for g in range(NG):
                    half16 = half_v[r, pl.ds(g * 16, 16)]
                    base = ((lane + g * 16) << 7) + half16
                    P = 16
                    vq = [
                        plsc.load_gather(stage, [zeros, base + d])
                        for d in range(P)
                    ]
                    for d in range(P, D):
                        blk_v[d - P, pl.ds(g * 16, 16)] = vq[d % P]
                        vq[d % P] = plsc.load_gather(stage, [zeros, base + d])
                    for d in range(D - P, D):
                        blk_v[d, pl.ds(g * 16, 16)] = vq[d % P]

                pltpu.sync_copy(blk_v, out_hbm.at[h, :, pl.ds(b0, BBLK)])
                return c

            lax.fori_loop(0, nh, unit_body, 0)

        def bb_body(bb, carry):
            b0 = pl.multiple_of(bbase + bb * BBLK, BBLK)

            def hblk_body(hblk, c):
                run_hblk(hblk * 8, 8, b0)
                return c

            lax.fori_loop(0, (HIST // 8), hblk_body, 0)
            run_hblk(HIST - HIST % 8, HIST % 8, b0)
            return carry

        lax.fori_loop(0, NBB, bb_body, 0)

    return gather


def kernel(input_ids, table):
    ids_t = jnp.transpose(input_ids).astype(jnp.int32)
    table2 = table.reshape(table.shape[0] // 2, 2 * D)
    out_t = _make_gather()(ids_t, table2)
    return jnp.transpose(out_t, (2, 0, 1))


# speedup vs baseline: 1.2087x; 1.0440x over previous
import functools

import jax
import jax.numpy as jnp
from jax import lax
from jax.experimental import pallas as pl
from jax.experimental.pallas import tpu as pltpu
from jax.experimental.pallas import tpu_sc as plsc

HIST = 50
BATCH = 16384
D = 64
NC, NS = 2, 16
NW = NC * NS
BBLK = 128
B_PER_W = BATCH // NW
NBB = B_PER_W // BBLK
NG = BBLK // 16
HB = BBLK // 2


def _make_gather():
    mesh = plsc.VectorSubcoreMesh(core_axis_name="c", subcore_axis_name="s")

    @functools.partial(
        pl.kernel,
        mesh=mesh,
        out_type=jax.ShapeDtypeStruct((HIST, D, BATCH), jnp.float32),
        scratch_types=[
            pltpu.VMEM((8, BBLK), jnp.int32),
            pltpu.VMEM((8, BBLK), jnp.int32),
            pltpu.VMEM((4, BBLK, 128), jnp.float32),
            pltpu.VMEM((2, D, BBLK), jnp.float32),
            pltpu.SemaphoreType.DMA,
            pltpu.SemaphoreType.DMA,
        ],
        compiler_params=pltpu.CompilerParams(
            use_tc_tiling_on_sc=True, needs_layout_passes=False
        ),
    )
    def gather(ids_hbm, table_hbm, out_hbm, pair_v, half_v, stage_v, blk_v,
               sem, wsem):
        wid = lax.axis_index("s") * NC + lax.axis_index("c")
        bbase = wid * B_PER_W
        lane = lax.iota(jnp.int32, 16)
        zeros = lane * 0

        def start_gather(r):
            buf = stage_v.at[r % 4]
            pltpu.async_copy(
                table_hbm.at[pair_v.at[r, pl.ds(0, HB)]],
                buf.at[pl.ds(0, HB)], sem,
            )
            pltpu.async_copy(
                table_hbm.at[pair_v.at[r, pl.ds(HB, HB)]],
                buf.at[pl.ds(HB, HB)], sem,
            )

        def wait_gather(r):
            pltpu.make_async_copy(
                table_hbm.at[pl.ds(0, BBLK)], stage_v.at[r % 4], sem
            ).wait()

        def wait_write():
            pltpu.make_async_copy(
                blk_v.at[0], out_hbm.at[0, :, pl.ds(0, BBLK)], wsem
            ).wait()

        def run_hblk(h0, nh, b0):
            pltpu.sync_copy(
                ids_hbm.at[pl.ds(h0, nh), pl.ds(b0, BBLK)],
                pair_v.at[pl.ds(0, nh)],
            )

            def prep_body(r, c):
                for g in range(NG):
                    ids16 = pair_v[r, pl.ds(g * 16, 16)]
                    half_v[r, pl.ds(g * 16, 16)] = (ids16 & 1) << 6
                    pair_v[r, pl.ds(g * 16, 16)] = ids16 >> 1
                return c

            lax.fori_loop(0, nh, prep_body, 0)
            for rr in range(min(3, nh)):
                start_gather(rr)

            def unit_body(r, c):
                h = h0 + r
                wait_gather(r)

                @pl.when(r < nh - 3)
                def _():
                    start_gather(r + 3)

                @pl.when(r >= 2)
                def _():
                    wait_write()

                stage = stage_v.at[r % 4]
                blk = blk_v.at[r % 2]
                for g in range(NG):
                    half16 = half_v[r, pl.ds(g * 16, 16)]
                    base = ((lane + g * 16) << 7) + half16
                    P = 16
                    vq = [
                        plsc.load_gather(stage, [zeros, base + d])
                        for d in range(P)
                    ]
                    for d in range(P, D):
                        blk[d - P, pl.ds(g * 16, 16)] = vq[d % P]
                        vq[d % P] = plsc.load_gather(stage, [zeros, base + d])
                    for d in range(D - P, D):
                        blk[d, pl.ds(g * 16, 16)] = vq[d % P]

                pltpu.async_copy(blk, out_hbm.at[h, :, pl.ds(b0, BBLK)], wsem)
                return c

            lax.fori_loop(0, nh, unit_body, 0)
            for _ in range(min(nh, 2)):
                wait_write()

        def bb_body(bb, carry):
            b0 = pl.multiple_of(bbase + bb * BBLK, BBLK)

            def hblk_body(hblk, c):
                run_hblk(hblk * 8, 8, b0)
                return c

            lax.fori_loop(0, (HIST // 8), hblk_body, 0)
            run_hblk(HIST - HIST % 8, HIST % 8, b0)
            return carry

        lax.fori_loop(0, NBB, bb_body, 0)

    return gather


def kernel(input_ids, table):
    ids_t = jnp.transpose(input_ids).astype(jnp.int32)
    table2 = table.reshape(table.shape[0] // 2, 2 * D)
    out_t = _make_gather()(ids_t, table2)
    return jnp.transpose(out_t, (2, 0, 1))


# speedup vs baseline: 1.3419x; 1.1102x over previous
import functools

import jax
import jax.numpy as jnp
from jax import lax
from jax.experimental import pallas as pl
from jax.experimental.pallas import tpu as pltpu
from jax.experimental.pallas import tpu_sc as plsc

D = 64
NC, NS = 2, 16
NW = NC * NS
IBLK = 128
K = 4
CH = K * IBLK
NBUF = 2


def _make_gather(n_rows: int):
    r_per_w = n_rows // NW
    n_chunks = r_per_w // CH
    n_outer = n_chunks // NBUF
    mesh = plsc.VectorSubcoreMesh(core_axis_name="c", subcore_axis_name="s")

    @functools.partial(
        pl.kernel,
        mesh=mesh,
        out_type=jax.ShapeDtypeStruct((n_rows, D), jnp.float32),
        scratch_types=[
            pltpu.VMEM((NBUF, K, IBLK), jnp.int32),
            pltpu.VMEM((NBUF, CH, D), jnp.float32),
            [pltpu.SemaphoreType.DMA] * NBUF,
            [pltpu.SemaphoreType.DMA] * NBUF,
        ],
        compiler_params=pltpu.CompilerParams(use_tc_tiling_on_sc=False),
    )
    def gather(idx_hbm, table_hbm, out_hbm, idx_v, rows_v, gsems, wsems):
        wid = lax.axis_index("s") * NC + lax.axis_index("c")
        row_base = wid * r_per_w
        iblk_base = row_base // IBLK

        def start_gather(c, b):
            ioff = pl.multiple_of(iblk_base + c * K, K)
            pltpu.sync_copy(idx_hbm.at[pl.ds(ioff, K)], idx_v.at[b])
            for j in range(K):
                pltpu.async_copy(
                    table_hbm.at[idx_v.at[b, j]],
                    rows_v.at[b, pl.ds(j * IBLK, IBLK)],
                    gsems[b],
                )

        def wait_gather(b):
            pltpu.make_async_copy(
                table_hbm.at[pl.ds(0, CH)], rows_v.at[b], gsems[b]
            ).wait()

        def start_write(c, b):
            off = pl.multiple_of(row_base + c * CH, CH)
            pltpu.async_copy(rows_v.at[b], out_hbm.at[pl.ds(off, CH)], wsems[b])

        def wait_write(b):
            pltpu.make_async_copy(
                rows_v.at[b], out_hbm.at[pl.ds(0, CH)], wsems[b]
            ).wait()

        for b in range(NBUF):
            start_gather(b, b)

        def outer_body(i, carry):
            c0 = i * NBUF
            for b in range(NBUF):
                wait_gather(b)
                start_write(c0 + b, b)
            for b in range(NBUF):
                wait_write(b)
                start_gather(c0 + b + NBUF, b)
            return carry

        lax.fori_loop(0, n_outer - 1, outer_body, 0)

        c0 = (n_outer - 1) * NBUF
        for b in range(NBUF):
            wait_gather(b)
            start_write(c0 + b, b)
        for b in range(NBUF):
            wait_write(b)

    return gather


def kernel(input_ids, table):
    b, h = input_ids.shape
    n = b * h
    idx2d = input_ids.reshape(n // IBLK, IBLK).astype(jnp.int32)
    out = _make_gather(n)(idx2d, table)
    return out.reshape(b, h, D)
